# baseline (speedup 1.0000x reference)
import jax
import jax.numpy as jnp
from jax.experimental import pallas as pl

N = 50000
NR = 32
G = 100
CUTOFF = 6.0


def _out_proj_kernel(g_ref, w_ref, o_ref):
    o_ref[...] = g_ref[...] @ w_ref[...]


def kernel(x, pos, edge_index, batch, W_in, b_in, W_msg_0, W_msg_1, W_node_0, W_node_1, W_out):
    src = edge_index[0]
    dst = edge_index[1]
    h = jax.nn.gelu(jnp.concatenate([x, pos], axis=1) @ W_in + b_in)
    rel = pos[dst] - pos[src]
    dist = jnp.sqrt(jnp.sum(rel * rel, axis=1) + 1e-8)
    centers = jnp.linspace(0.0, CUTOFF, NR)
    rbf = jnp.exp(-((dist[:, None] - centers[None, :]) ** 2) / 0.5)
    env = 0.5 * (jnp.cos(jnp.pi * jnp.clip(dist / CUTOFF, 0.0, 1.0)) + 1.0)
    rbf = rbf * env[:, None]
    for W_msg, W_node in ((W_msg_0, W_node_0), (W_msg_1, W_node_1)):
        m = jax.nn.gelu(jnp.concatenate([h[src], rbf], axis=1) @ W_msg)
        agg = jax.ops.segment_sum(m, dst, num_segments=N)
        h = h + jax.nn.gelu(jnp.concatenate([h, agg], axis=1) @ W_node)
    g = jax.ops.segment_sum(h, batch, num_segments=G)
    out = pl.pallas_call(
        _out_proj_kernel,
        out_shape=jax.ShapeDtypeStruct((G, W_out.shape[1]), jnp.float32),
    )(g, W_out)
    return out


# speedup vs baseline: 1.6024x; 1.6024x over previous
import jax
import jax.numpy as jnp
from jax import lax
from jax.experimental import pallas as pl
from jax.experimental.pallas import tpu as pltpu
from jax.experimental.pallas import tpu_sc as plsc

N = 50000
E = 800000
D = 64
NR = 32
BOT = 128
G = 100
CUTOFF = 6.0

NC = 2
NS = 16
L = 16

E_PAD = 802816
PAD = E_PAD - E
ROWS = E_PAD // 128

NPAD = 53248
ZROWS = NPAD // NS
BN = 1000
NB = N // BN

EPT = E_PAD // NS
MAC = 896
MT = MAC // 128
NMAC = EPT // MAC

EPW = E_PAD // (NC * NS)
GMAC = 1792
GROWS = GMAC // 128
NGMAC = EPW // GMAC

_SQ2OPI_X2 = 1.5957691216057308
_GELU_C = 0.044715

_SC_PARAMS = pltpu.CompilerParams(use_tc_tiling_on_sc=False,
                                  needs_layout_passes=False)


def _gelu16(x):
    u = x + _GELU_C * (x * x * x)
    return x / (1.0 + jnp.exp(-_SQ2OPI_X2 * u))


def _sc_d2_body(pos16_hbm, src_hbm, dst_hbm, d2_hbm,
                idxs_v, idxd_v, ps_v, pd_v, d2_v, gsem):
    c = lax.axis_index("c")
    s = lax.axis_index("s")
    wid = s * NC + c
    iota = lax.broadcasted_iota(jnp.int32, (L,), 0)
    c0 = jnp.zeros((L,), jnp.int32)
    c1 = jnp.full((L,), 1, jnp.int32)
    c2 = jnp.full((L,), 2, jnp.int32)

    def macro(mg, carry):
        rb = wid * (NGMAC * GROWS) + mg * GROWS
        pltpu.sync_copy(src_hbm.at[pl.ds(rb, GROWS)], idxs_v)
        pltpu.sync_copy(dst_hbm.at[pl.ds(rb, GROWS)], idxd_v)
        descs = []
        for j in range(GROWS):
            descs.append(pltpu.async_copy(
                pos16_hbm.at[idxs_v.at[j]],
                ps_v.at[pl.ds(j * 128, 128)], gsem))
            descs.append(pltpu.async_copy(
                pos16_hbm.at[idxd_v.at[j]],
                pd_v.at[pl.ds(j * 128, 128)], gsem))
        for d in descs:
            d.wait()
        for j in range(GROWS * 8):
            rows = iota + j * L
            dx = (plsc.load_gather(pd_v, [rows, c0])
                  - plsc.load_gather(ps_v, [rows, c0]))
            dy = (plsc.load_gather(pd_v, [rows, c1])
                  - plsc.load_gather(ps_v, [rows, c1]))
            dz = (plsc.load_gather(pd_v, [rows, c2])
                  - plsc.load_gather(ps_v, [rows, c2]))
            d2_v[pl.ds(j * L, L)] = dx * dx + dy * dy + dz * dz
        pltpu.sync_copy(d2_v, d2_hbm.at[pl.ds(wid * EPW + mg * GMAC, GMAC)])
        return carry

    lax.fori_loop(0, NGMAC, macro, 0)


def _sc_d2(pos16, srcp, dstg):
    mesh = plsc.VectorSubcoreMesh(core_axis_name="c", subcore_axis_name="s")
    return pl.kernel(
        _sc_d2_body,
        out_type=jax.ShapeDtypeStruct((E_PAD,), jnp.float32),
        mesh=mesh,
        scratch_types=[
            pltpu.VMEM((GROWS, 128), jnp.int32),
            pltpu.VMEM((GROWS, 128), jnp.int32),
            pltpu.VMEM((GMAC, 16), jnp.float32),
            pltpu.VMEM((GMAC, 16), jnp.float32),
            pltpu.VMEM((GMAC,), jnp.float32),
            pltpu.SemaphoreType.DMA,
        ],
        compiler_params=_SC_PARAMS,
    )(pos16, srcp, dstg)


def _tc_rbf_kernel(d2_ref, b0t_ref, b1t_ref, ew0_ref, ew1_ref):
    c_col = (lax.broadcasted_iota(jnp.int32, (NR, 1), 0).astype(jnp.float32)
             * (CUTOFF / (NR - 1)))
    p0, p1 = [], []
    for j in range(8):
        d2row = d2_ref[j:j + 1, :]
        dist = jnp.sqrt(d2row + 1e-8)
        t = jnp.clip(dist * (1.0 / CUTOFF), 0.0, 1.0)
        env = 0.5 * (jnp.cos(jnp.pi * t) + 1.0)
        diff = dist - c_col
        rbft = jnp.exp(-(diff * diff) * 2.0) * env
        p0.append(b0t_ref[...] @ rbft)
        p1.append(b1t_ref[...] @ rbft)
    for pieces, ref in ((p0, ew0_ref), (p1, ew1_ref)):
        bands = []
        for k in range(8):
            tiles = [p[8 * k:8 * k + 8, :].reshape(1, 8, 128) for p in pieces]
            bands.append(jnp.concatenate(tiles, axis=0).reshape(1, 8, 8, 128))
        ref[...] = jnp.concatenate(bands, axis=0)


def _tc_rbf(d2m, b0t, b1t):
    return pl.pallas_call(
        _tc_rbf_kernel,
        grid=(ROWS // 8,),
        in_specs=[
            pl.BlockSpec((8, 128), lambda i: (i, 0)),
            pl.BlockSpec((D, NR), lambda i: (0, 0)),
            pl.BlockSpec((D, NR), lambda i: (0, 0)),
        ],
        out_specs=[
            pl.BlockSpec((8, 8, 8, 128), lambda i: (0, i, 0, 0)),
            pl.BlockSpec((8, 8, 8, 128), lambda i: (0, i, 0, 0)),
        ],
        out_shape=[
            jax.ShapeDtypeStruct((8, ROWS, 8, 128), jnp.float32),
            jax.ShapeDtypeStruct((8, ROWS, 8, 128), jnp.float32),
        ],
    )(d2m, b0t, b1t)


def _tc_embed_kernel(x_ref, pos_ref, win_ref, bin_ref, a0_ref, h_ref, ha_ref):
    xp = jnp.concatenate([x_ref[...], pos_ref[...]], axis=1)
    h = jax.nn.gelu(xp @ win_ref[...] + bin_ref[...])
    h_ref[...] = h
    ha = h @ a0_ref[...]
    for q in range(4):
        ha_ref[q] = ha[:, 16 * q:16 * q + 16]


def _tc_embed(x, pos, w_in, b_in, a0):
    return pl.pallas_call(
        _tc_embed_kernel,
        grid=(NB,),
        in_specs=[
            pl.BlockSpec((BN, 1), lambda i: (i, 0)),
            pl.BlockSpec((BN, 3), lambda i: (i, 0)),
            pl.BlockSpec((4, D), lambda i: (0, 0)),
            pl.BlockSpec((1, D), lambda i: (0, 0)),
            pl.BlockSpec((D, D), lambda i: (0, 0)),
        ],
        out_specs=[
            pl.BlockSpec((BN, D), lambda i: (i, 0)),
            pl.BlockSpec((4, BN, 16), lambda i: (0, i, 0)),
        ],
        out_shape=[
            jax.ShapeDtypeStruct((N, D), jnp.float32),
            jax.ShapeDtypeStruct((4, N, 16), jnp.float32),
        ],
    )(x, pos, w_in, b_in, a0)


def _sc_layer_body(src_hbm, dst_hbm, ew_hbm, ha_hbm, agg_hbm,
                   idxs_v, idxd_v, gha_v, ew_v, zb_v, acc_sh,
                   gsem, ssem, esem):
    c = lax.axis_index("c")
    s = lax.axis_index("s")
    iota = lax.broadcasted_iota(jnp.int32, (L,), 0)

    zeros16 = jnp.zeros((L,), jnp.float32)
    for j in range(128):
        zb_v[j, pl.ds(0, L)] = zeros16

    tail = N - 15 * ZROWS

    for p in range(2):
        def zloop(zi, carry):
            pltpu.sync_copy(zb_v, acc_sh.at[pl.ds(s * ZROWS + zi * 128, 128)])
            return carry
        lax.fori_loop(0, ZROWS // 128, zloop, 0)
        plsc.subcore_barrier()

        qoff = jnp.broadcast_to((2 * c + p) * N, (L,)).astype(jnp.int32)

        def macro(mg, carry):
            rb = s * (NMAC * MT) + mg * MT
            pltpu.sync_copy(src_hbm.at[pl.ds(rb, MT)], idxs_v)
            pltpu.sync_copy(dst_hbm.at[pl.ds(rb, MT)], idxd_v)
            for j in range(MT):
                for k in range(8):
                    v = idxs_v[j, pl.ds(k * L, L)]
                    idxs_v[j, pl.ds(k * L, L)] = v + qoff
            descs = [pltpu.async_copy(ha_hbm.at[idxs_v.at[j]],
                                      gha_v.at[pl.ds(j * 128, 128)], gsem)
                     for j in range(MT)]
            for b in range(2):
                descs.append(pltpu.async_copy(
                    ew_hbm.at[4 * c + 2 * p + b, pl.ds(rb, MT)],
                    ew_v.at[b], esem))
            for d in descs:
                d.wait()

            def group(g, carry2):
                e0 = g * L
                t = lax.shift_right_logical(g, 3)
                lane0 = jnp.bitwise_and(g, 7) * L
                rows = iota + e0
                for b in range(2):
                    for j in range(8):
                        dcol = jnp.full((L,), 8 * b + j, jnp.int32)
                        ew16 = ew_v[b, t, j, pl.ds(lane0, L)]
                        ga16 = plsc.load_gather(gha_v, [rows, dcol])
                        m16 = _gelu16(ew16 + ga16)
                        plsc.store_scatter(gha_v, [rows, dcol], m16)
                return carry2
            lax.fori_loop(0, MAC // L, group, 0)

            sdescs = [pltpu.async_copy(gha_v.at[pl.ds(j * 128, 128)],
                                       acc_sh.at[idxd_v.at[j]], ssem, add=True)
                      for j in range(MT)]
            for d in sdescs:
                d.wait()
            return carry

        lax.fori_loop(0, NMAC, macro, 0)
        plsc.subcore_barrier()

        q = 2 * c + p

        @pl.when(s < 15)
        def _():
            pltpu.sync_copy(acc_sh.at[pl.ds(s * ZROWS, ZROWS)],
                            agg_hbm.at[q, pl.ds(s * ZROWS, ZROWS)])

        @pl.when(s == 15)
        def _():
            pltpu.sync_copy(acc_sh.at[pl.ds(15 * ZROWS, tail)],
                            agg_hbm.at[q, pl.ds(15 * ZROWS, tail)])


def _sc_layer(srcp, dstp, ew, ha4n):
    mesh = plsc.VectorSubcoreMesh(core_axis_name="c", subcore_axis_name="s")
    return pl.kernel(
        _sc_layer_body,
        out_type=jax.ShapeDtypeStruct((4, N, 16), jnp.float32),
        mesh=mesh,
        scratch_types=[
            pltpu.VMEM((MT, 128), jnp.int32),
            pltpu.VMEM((MT, 128), jnp.int32),
            pltpu.VMEM((MAC, 16), jnp.float32),
            pltpu.VMEM((2, MT, 8, 128), jnp.float32),
            pltpu.VMEM((128, 16), jnp.float32),
            pltpu.VMEM_SHARED((NPAD, 16), jnp.float32),
            pltpu.SemaphoreType.DMA,
            pltpu.SemaphoreType.DMA,
            pltpu.SemaphoreType.DMA,
        ],
        compiler_params=_SC_PARAMS,
    )(srcp, dstp, ew, ha4n)


def _tc_update_kernel(h_ref, agg_ref, wn_ref, a_ref, h1_ref, ha_ref):
    aggf = jnp.concatenate([agg_ref[q] for q in range(4)], axis=1)
    z = jnp.concatenate([h_ref[...], aggf], axis=1)
    h1 = h_ref[...] + jax.nn.gelu(z @ wn_ref[...])
    h1_ref[...] = h1
    ha = h1 @ a_ref[...]
    for q in range(4):
        ha_ref[q] = ha[:, 16 * q:16 * q + 16]


def _tc_update(h, agg, w_node, a_next):
    return pl.pallas_call(
        _tc_update_kernel,
        grid=(NB,),
        in_specs=[
            pl.BlockSpec((BN, D), lambda i: (i, 0)),
            pl.BlockSpec((4, BN, 16), lambda i: (0, i, 0)),
            pl.BlockSpec((2 * D, D), lambda i: (0, 0)),
            pl.BlockSpec((D, D), lambda i: (0, 0)),
        ],
        out_specs=[
            pl.BlockSpec((BN, D), lambda i: (i, 0)),
            pl.BlockSpec((4, BN, 16), lambda i: (0, i, 0)),
        ],
        out_shape=[
            jax.ShapeDtypeStruct((N, D), jnp.float32),
            jax.ShapeDtypeStruct((4, N, 16), jnp.float32),
        ],
    )(h, agg, w_node, a_next)


def _tc_final_kernel(h_ref, agg_ref, wn_ref, batch_ref, wout_ref, out_ref,
                     gacc_ref):
    i = pl.program_id(0)

    @pl.when(i == 0)
    def _():
        gacc_ref[...] = jnp.zeros_like(gacc_ref)

    aggf = jnp.concatenate([agg_ref[q] for q in range(4)], axis=1)
    z = jnp.concatenate([h_ref[...], aggf], axis=1)
    h2 = h_ref[...] + jax.nn.gelu(z @ wn_ref[...])
    gid = lax.broadcasted_iota(jnp.int32, (BN, 128), 1)
    oh = (batch_ref[...] == gid).astype(jnp.float32)
    gacc_ref[...] += lax.dot_general(
        oh, h2, (((0,), (0,)), ((), ())))

    @pl.when(i == NB - 1)
    def _():
        out_ref[...] = gacc_ref[...] @ wout_ref[...]


def _tc_final(h, agg, w_node, batch_col, w_out):
    return pl.pallas_call(
        _tc_final_kernel,
        grid=(NB,),
        in_specs=[
            pl.BlockSpec((BN, D), lambda i: (i, 0)),
            pl.BlockSpec((4, BN, 16), lambda i: (0, i, 0)),
            pl.BlockSpec((2 * D, D), lambda i: (0, 0)),
            pl.BlockSpec((BN, 1), lambda i: (i, 0)),
            pl.BlockSpec((D, BOT), lambda i: (0, 0)),
        ],
        out_specs=pl.BlockSpec((128, BOT), lambda i: (0, 0)),
        out_shape=jax.ShapeDtypeStruct((128, BOT), jnp.float32),
        scratch_shapes=[pltpu.VMEM((128, D), jnp.float32)],
    )(h, agg, w_node, batch_col, w_out)


def kernel(x, pos, edge_index, batch, W_in, b_in, W_msg_0, W_msg_1,
           W_node_0, W_node_1, W_out):
    src = edge_index[0]
    dst = edge_index[1]
    zpad = jnp.zeros((PAD,), jnp.int32)
    srcp = jnp.concatenate([src, zpad]).reshape(ROWS, 128)
    dstg = jnp.concatenate([dst, zpad]).reshape(ROWS, 128)
    dstp = jnp.concatenate(
        [dst, N + jnp.arange(PAD, dtype=jnp.int32)]).reshape(ROWS, 128)
    pos16 = jnp.pad(pos, ((0, 0), (0, 13)))

    a0, b0t = W_msg_0[:D], W_msg_0[D:].T
    a1, b1t = W_msg_1[:D], W_msg_1[D:].T

    d2 = _sc_d2(pos16, srcp, dstg)
    ew0, ew1 = _tc_rbf(d2.reshape(ROWS, 128), b0t, b1t)
    h0, ha0 = _tc_embed(x, pos, W_in, b_in.reshape(1, D), a0)

    agg0 = _sc_layer(srcp, dstp, ew0, ha0.reshape(4 * N, 16))
    h1, ha1 = _tc_update(h0, agg0, W_node_0, a1)
    agg1 = _sc_layer(srcp, dstp, ew1, ha1.reshape(4 * N, 16))
    out128 = _tc_final(h1, agg1, W_node_1, batch.reshape(N, 1), W_out)
    return out128[:G]


# speedup vs baseline: 1.8083x; 1.1284x over previous
import jax
import jax.numpy as jnp
from jax import lax
from jax.experimental import pallas as pl
from jax.experimental.pallas import tpu as pltpu
from jax.experimental.pallas import tpu_sc as plsc

N = 50000
E = 800000
D = 64
NR = 32
BOT = 128
G = 100
CUTOFF = 6.0

NC = 2
NS = 16
L = 16

E_PAD = 802816
PAD = E_PAD - E
ROWS = E_PAD // 128

NPAD = 53248
ZROWS = NPAD // NS
BN = 1000
NB = N // BN

EPT = E_PAD // NS
MAC = 1792
MT = MAC // 128
NMAC = EPT // MAC

EPW = E_PAD // (NC * NS)
GMAC = 1792
GROWS = GMAC // 128
NGMAC = EPW // GMAC

_SQ2OPI_X2 = 1.5957691216057308
_GELU_C = 0.044715

_SC_PARAMS = pltpu.CompilerParams(use_tc_tiling_on_sc=False,
                                  needs_layout_passes=False)


def _gelu16(x):
    u = x + _GELU_C * (x * x * x)
    return x / (1.0 + jnp.exp(-_SQ2OPI_X2 * u))


def _sc_d2_body(pos16_hbm, src_hbm, dst_hbm, d2_hbm,
                idxs_v, idxd_v, ps_v, pd_v, d2_v, gsem):
    c = lax.axis_index("c")
    s = lax.axis_index("s")
    wid = s * NC + c
    iota = lax.broadcasted_iota(jnp.int32, (L,), 0)
    c0 = jnp.zeros((L,), jnp.int32)
    c1 = jnp.full((L,), 1, jnp.int32)
    c2 = jnp.full((L,), 2, jnp.int32)

    def macro(mg, carry):
        rb = wid * (NGMAC * GROWS) + mg * GROWS
        pltpu.sync_copy(src_hbm.at[pl.ds(rb, GROWS)], idxs_v)
        pltpu.sync_copy(dst_hbm.at[pl.ds(rb, GROWS)], idxd_v)
        descs = []
        for j in range(GROWS):
            descs.append(pltpu.async_copy(
                pos16_hbm.at[idxs_v.at[j]],
                ps_v.at[pl.ds(j * 128, 128)], gsem))
            descs.append(pltpu.async_copy(
                pos16_hbm.at[idxd_v.at[j]],
                pd_v.at[pl.ds(j * 128, 128)], gsem))
        for d in descs:
            d.wait()
        for j in range(GROWS * 8):
            rows = iota + j * L
            dx = (plsc.load_gather(pd_v, [rows, c0])
                  - plsc.load_gather(ps_v, [rows, c0]))
            dy = (plsc.load_gather(pd_v, [rows, c1])
                  - plsc.load_gather(ps_v, [rows, c1]))
            dz = (plsc.load_gather(pd_v, [rows, c2])
                  - plsc.load_gather(ps_v, [rows, c2]))
            d2_v[pl.ds(j * L, L)] = dx * dx + dy * dy + dz * dz
        pltpu.sync_copy(d2_v, d2_hbm.at[pl.ds(wid * EPW + mg * GMAC, GMAC)])
        return carry

    lax.fori_loop(0, NGMAC, macro, 0)


def _sc_d2(pos16, srcp, dstg):
    mesh = plsc.VectorSubcoreMesh(core_axis_name="c", subcore_axis_name="s")
    return pl.kernel(
        _sc_d2_body,
        out_type=jax.ShapeDtypeStruct((E_PAD,), jnp.float32),
        mesh=mesh,
        scratch_types=[
            pltpu.VMEM((GROWS, 128), jnp.int32),
            pltpu.VMEM((GROWS, 128), jnp.int32),
            pltpu.VMEM((GMAC, 16), jnp.float32),
            pltpu.VMEM((GMAC, 16), jnp.float32),
            pltpu.VMEM((GMAC,), jnp.float32),
            pltpu.SemaphoreType.DMA,
        ],
        compiler_params=_SC_PARAMS,
    )(pos16, srcp, dstg)


def _tc_rbf_kernel(d2_ref, b0t_ref, b1t_ref, ew0_ref, ew1_ref):
    c_col = (lax.broadcasted_iota(jnp.int32, (NR, 1), 0).astype(jnp.float32)
             * (CUTOFF / (NR - 1)))
    p0, p1 = [], []
    for j in range(8):
        d2row = d2_ref[j:j + 1, :]
        dist = jnp.sqrt(d2row + 1e-8)
        t = jnp.clip(dist * (1.0 / CUTOFF), 0.0, 1.0)
        env = 0.5 * (jnp.cos(jnp.pi * t) + 1.0)
        diff = dist - c_col
        rbft = jnp.exp(-(diff * diff) * 2.0) * env
        p0.append(b0t_ref[...] @ rbft)
        p1.append(b1t_ref[...] @ rbft)
    for pieces, ref in ((p0, ew0_ref), (p1, ew1_ref)):
        bands = []
        for k in range(8):
            tiles = [p[8 * k:8 * k + 8, :].reshape(1, 8, 128) for p in pieces]
            bands.append(jnp.concatenate(tiles, axis=0).reshape(1, 8, 8, 128))
        ref[...] = jnp.concatenate(bands, axis=0)


def _tc_rbf(d2m, b0t, b1t):
    return pl.pallas_call(
        _tc_rbf_kernel,
        grid=(ROWS // 8,),
        in_specs=[
            pl.BlockSpec((8, 128), lambda i: (i, 0)),
            pl.BlockSpec((D, NR), lambda i: (0, 0)),
            pl.BlockSpec((D, NR), lambda i: (0, 0)),
        ],
        out_specs=[
            pl.BlockSpec((8, 8, 8, 128), lambda i: (0, i, 0, 0)),
            pl.BlockSpec((8, 8, 8, 128), lambda i: (0, i, 0, 0)),
        ],
        out_shape=[
            jax.ShapeDtypeStruct((8, ROWS, 8, 128), jnp.float32),
            jax.ShapeDtypeStruct((8, ROWS, 8, 128), jnp.float32),
        ],
    )(d2m, b0t, b1t)


def _tc_embed_kernel(x_ref, pos_ref, win_ref, bin_ref, a0_ref, h_ref, ha_ref):
    xp = jnp.concatenate([x_ref[...], pos_ref[...]], axis=1)
    h = jax.nn.gelu(xp @ win_ref[...] + bin_ref[...])
    h_ref[...] = h
    ha = h @ a0_ref[...]
    for q in range(4):
        ha_ref[q] = ha[:, 16 * q:16 * q + 16]


def _tc_embed(x, pos, w_in, b_in, a0):
    return pl.pallas_call(
        _tc_embed_kernel,
        grid=(NB,),
        in_specs=[
            pl.BlockSpec((BN, 1), lambda i: (i, 0)),
            pl.BlockSpec((BN, 3), lambda i: (i, 0)),
            pl.BlockSpec((4, D), lambda i: (0, 0)),
            pl.BlockSpec((1, D), lambda i: (0, 0)),
            pl.BlockSpec((D, D), lambda i: (0, 0)),
        ],
        out_specs=[
            pl.BlockSpec((BN, D), lambda i: (i, 0)),
            pl.BlockSpec((4, BN, 16), lambda i: (0, i, 0)),
        ],
        out_shape=[
            jax.ShapeDtypeStruct((N, D), jnp.float32),
            jax.ShapeDtypeStruct((4, N, 16), jnp.float32),
        ],
    )(x, pos, w_in, b_in, a0)


def _sc_layer_body(src_hbm, dst_hbm, ew_hbm, ha_hbm, agg_hbm,
                   idxs_v, idxd_v, gha_v, ew_v, zb_v, acc_sh,
                   gsem, ssem, esem):
    c = lax.axis_index("c")
    s = lax.axis_index("s")
    iota = lax.broadcasted_iota(jnp.int32, (L,), 0)

    zeros16 = jnp.zeros((L,), jnp.float32)
    for j in range(128):
        zb_v[j, pl.ds(0, L)] = zeros16

    tail = N - 15 * ZROWS

    for p in range(2):
        def zloop(zi, carry):
            pltpu.sync_copy(zb_v, acc_sh.at[pl.ds(s * ZROWS + zi * 128, 128)])
            return carry
        lax.fori_loop(0, ZROWS // 128, zloop, 0)
        plsc.subcore_barrier()

        qoff = jnp.broadcast_to((2 * c + p) * N, (L,)).astype(jnp.int32)

        def macro(mg, carry):
            rb = s * (NMAC * MT) + mg * MT
            pltpu.sync_copy(src_hbm.at[pl.ds(rb, MT)], idxs_v)
            pltpu.sync_copy(dst_hbm.at[pl.ds(rb, MT)], idxd_v)
            for j in range(MT):
                for k in range(8):
                    v = idxs_v[j, pl.ds(k * L, L)]
                    idxs_v[j, pl.ds(k * L, L)] = v + qoff
            descs = [pltpu.async_copy(ha_hbm.at[idxs_v.at[j]],
                                      gha_v.at[pl.ds(j * 128, 128)], gsem)
                     for j in range(MT)]
            for b in range(2):
                descs.append(pltpu.async_copy(
                    ew_hbm.at[4 * c + 2 * p + b, pl.ds(rb, MT)],
                    ew_v.at[:, pl.ds(8 * b, 8), :], esem))
            for d in descs:
                d.wait()

            def group(g, carry2):
                e0 = g * L
                t = lax.shift_right_logical(g, 3)
                lane0 = jnp.bitwise_and(g, 7) * L
                rows = iota + e0
                lanes = iota + lane0
                tfull = jnp.broadcast_to(t, (L,)).astype(jnp.int32)
                for k in range(L):
                    col = jnp.bitwise_and(iota + k, L - 1)
                    ew16 = plsc.load_gather(ew_v, [tfull, col, lanes])
                    ga16 = plsc.load_gather(gha_v, [rows, col])
                    m16 = _gelu16(ew16 + ga16)
                    plsc.store_scatter(gha_v, [rows, col], m16)
                return carry2
            lax.fori_loop(0, MAC // L, group, 0)

            sdescs = [pltpu.async_copy(gha_v.at[pl.ds(j * 128, 128)],
                                       acc_sh.at[idxd_v.at[j]], ssem, add=True)
                      for j in range(MT)]
            for d in sdescs:
                d.wait()
            return carry

        lax.fori_loop(0, NMAC, macro, 0)
        plsc.subcore_barrier()

        q = 2 * c + p

        @pl.when(s < 15)
        def _():
            pltpu.sync_copy(acc_sh.at[pl.ds(s * ZROWS, ZROWS)],
                            agg_hbm.at[q, pl.ds(s * ZROWS, ZROWS)])

        @pl.when(s == 15)
        def _():
            pltpu.sync_copy(acc_sh.at[pl.ds(15 * ZROWS, tail)],
                            agg_hbm.at[q, pl.ds(15 * ZROWS, tail)])


def _sc_layer(srcp, dstp, ew, ha4n):
    mesh = plsc.VectorSubcoreMesh(core_axis_name="c", subcore_axis_name="s")
    return pl.kernel(
        _sc_layer_body,
        out_type=jax.ShapeDtypeStruct((4, N, 16), jnp.float32),
        mesh=mesh,
        scratch_types=[
            pltpu.VMEM((MT, 128), jnp.int32),
            pltpu.VMEM((MT, 128), jnp.int32),
            pltpu.VMEM((MAC, 16), jnp.float32),
            pltpu.VMEM((MT, 16, 128), jnp.float32),
            pltpu.VMEM((128, 16), jnp.float32),
            pltpu.VMEM_SHARED((NPAD, 16), jnp.float32),
            pltpu.SemaphoreType.DMA,
            pltpu.SemaphoreType.DMA,
            pltpu.SemaphoreType.DMA,
        ],
        compiler_params=_SC_PARAMS,
    )(srcp, dstp, ew, ha4n)


def _tc_update_kernel(h_ref, agg_ref, wn_ref, a_ref, h1_ref, ha_ref):
    aggf = jnp.concatenate([agg_ref[q] for q in range(4)], axis=1)
    z = jnp.concatenate([h_ref[...], aggf], axis=1)
    h1 = h_ref[...] + jax.nn.gelu(z @ wn_ref[...])
    h1_ref[...] = h1
    ha = h1 @ a_ref[...]
    for q in range(4):
        ha_ref[q] = ha[:, 16 * q:16 * q + 16]


def _tc_update(h, agg, w_node, a_next):
    return pl.pallas_call(
        _tc_update_kernel,
        grid=(NB,),
        in_specs=[
            pl.BlockSpec((BN, D), lambda i: (i, 0)),
            pl.BlockSpec((4, BN, 16), lambda i: (0, i, 0)),
            pl.BlockSpec((2 * D, D), lambda i: (0, 0)),
            pl.BlockSpec((D, D), lambda i: (0, 0)),
        ],
        out_specs=[
            pl.BlockSpec((BN, D), lambda i: (i, 0)),
            pl.BlockSpec((4, BN, 16), lambda i: (0, i, 0)),
        ],
        out_shape=[
            jax.ShapeDtypeStruct((N, D), jnp.float32),
            jax.ShapeDtypeStruct((4, N, 16), jnp.float32),
        ],
    )(h, agg, w_node, a_next)


def _tc_final_kernel(h_ref, agg_ref, wn_ref, batch_ref, wout_ref, out_ref,
                     gacc_ref):
    i = pl.program_id(0)

    @pl.when(i == 0)
    def _():
        gacc_ref[...] = jnp.zeros_like(gacc_ref)

    aggf = jnp.concatenate([agg_ref[q] for q in range(4)], axis=1)
    z = jnp.concatenate([h_ref[...], aggf], axis=1)
    h2 = h_ref[...] + jax.nn.gelu(z @ wn_ref[...])
    gid = lax.broadcasted_iota(jnp.int32, (BN, 128), 1)
    oh = (batch_ref[...] == gid).astype(jnp.float32)
    gacc_ref[...] += lax.dot_general(
        oh, h2, (((0,), (0,)), ((), ())))

    @pl.when(i == NB - 1)
    def _():
        out_ref[...] = gacc_ref[...] @ wout_ref[...]


def _tc_final(h, agg, w_node, batch_col, w_out):
    return pl.pallas_call(
        _tc_final_kernel,
        grid=(NB,),
        in_specs=[
            pl.BlockSpec((BN, D), lambda i: (i, 0)),
            pl.BlockSpec((4, BN, 16), lambda i: (0, i, 0)),
            pl.BlockSpec((2 * D, D), lambda i: (0, 0)),
            pl.BlockSpec((BN, 1), lambda i: (i, 0)),
            pl.BlockSpec((D, BOT), lambda i: (0, 0)),
        ],
        out_specs=pl.BlockSpec((128, BOT), lambda i: (0, 0)),
        out_shape=jax.ShapeDtypeStruct((128, BOT), jnp.float32),
        scratch_shapes=[pltpu.VMEM((128, D), jnp.float32)],
    )(h, agg, w_node, batch_col, w_out)


def kernel(x, pos, edge_index, batch, W_in, b_in, W_msg_0, W_msg_1,
           W_node_0, W_node_1, W_out):
    src = edge_index[0]
    dst = edge_index[1]
    zpad = jnp.zeros((PAD,), jnp.int32)
    srcp = jnp.concatenate([src, zpad]).reshape(ROWS, 128)
    dstg = jnp.concatenate([dst, zpad]).reshape(ROWS, 128)
    dstp = jnp.concatenate(
        [dst, N + jnp.arange(PAD, dtype=jnp.int32)]).reshape(ROWS, 128)
    pos16 = jnp.pad(pos, ((0, 0), (0, 13)))

    a0, b0t = W_msg_0[:D], W_msg_0[D:].T
    a1, b1t = W_msg_1[:D], W_msg_1[D:].T

    d2 = _sc_d2(pos16, srcp, dstg)
    ew0, ew1 = _tc_rbf(d2.reshape(ROWS, 128), b0t, b1t)
    h0, ha0 = _tc_embed(x, pos, W_in, b_in.reshape(1, D), a0)

    agg0 = _sc_layer(srcp, dstp, ew0, ha0.reshape(4 * N, 16))
    h1, ha1 = _tc_update(h0, agg0, W_node_0, a1)
    agg1 = _sc_layer(srcp, dstp, ew1, ha1.reshape(4 * N, 16))
    out128 = _tc_final(h1, agg1, W_node_1, batch.reshape(N, 1), W_out)
    return out128[:G]
